# Initial kernel scaffold; baseline (speedup 1.0000x reference)
import functools

import jax
import jax.numpy as jnp
from jax import lax
from jax.experimental import pallas as pl
from jax.experimental.pallas import tpu as pltpu
from jax.experimental.pallas import tpu_sc as plsc

VOCAB = 1000000
D = 32
O = 6
B = 4096
H = 200

NC = 2
NS = 16
L = 16
NW = NC * NS
EPW = B // NW
GSUB = 40
NG = H // GSUB

_mesh = plsc.VectorSubcoreMesh(
    core_axis_name="c", subcore_axis_name="s", num_cores=NC, num_subcores=NS)


@functools.partial(
    pl.kernel,
    out_type=jax.ShapeDtypeStruct((B, O), jnp.float32),
    mesh=_mesh,
    scratch_types=dict(
        idx_v=pltpu.VMEM((EPW * H,), jnp.int32),
        rows_a=pltpu.VMEM((H, D), jnp.float32),
        rows_b=pltpu.VMEM((H, D), jnp.float32),
        acc=pltpu.VMEM((EPW, D), jnp.float32),
        wv=pltpu.VMEM((O, D, L), jnp.float32),
        bv=pltpu.VMEM((O, L), jnp.float32),
        out_v=pltpu.VMEM((EPW, O), jnp.float32),
        sem_a=pltpu.SemaphoreType.DMA,
        sem_b=pltpu.SemaphoreType.DMA,
    ),
)
def _sc_kernel(x_hbm, table_hbm, w_hbm, b_hbm, out_hbm,
               idx_v, rows_a, rows_b, acc, wv, bv, out_v, sem_a, sem_b):
  wid = lax.axis_index("s") * NC + lax.axis_index("c")
  base = wid * EPW

  pltpu.sync_copy(x_hbm.at[pl.ds(base * H, EPW * H)], idx_v)
  pltpu.sync_copy(w_hbm, wv)
  pltpu.sync_copy(b_hbm, bv)

  def fire(e, buf, sem):
    for j in range(NG):
      off = pl.multiple_of(e * H + j * GSUB, 8)
      pltpu.async_copy(
          table_hbm.at[idx_v.at[pl.ds(off, GSUB)]],
          buf.at[pl.ds(j * GSUB, GSUB)], sem)

  def drain(buf, sem):
    pltpu.make_async_copy(table_hbm.at[pl.ds(0, H)], buf, sem).wait()

  zeros = jnp.zeros((L,), jnp.float32)

  def reduce(e, buf):
    def body(i, carry):
      a0, a1 = carry
      for k in range(8):
        r = i * 8 + k
        a0 = a0 + buf[r, pl.ds(0, L)]
        a1 = a1 + buf[r, pl.ds(L, L)]
      return a0, a1
    a0, a1 = lax.fori_loop(0, H // 8, body, (zeros, zeros))
    acc[e, pl.ds(0, L)] = a0
    acc[e, pl.ds(L, L)] = a1

  fire(0, rows_a, sem_a)

  def pipe(ee, _):
    e0 = ee * 2
    drain(rows_a, sem_a)
    fire(e0 + 1, rows_b, sem_b)
    reduce(e0, rows_a)
    drain(rows_b, sem_b)

    @pl.when(e0 + 2 < EPW)
    def _():
      fire(e0 + 2, rows_a, sem_a)

    reduce(e0 + 1, rows_b)
    return 0

  lax.fori_loop(0, EPW // 2, pipe, 0)

  iota = lax.iota(jnp.int32, L)
  for g in range(EPW // L):
    bvec = g * L + iota
    pdt = [plsc.load_gather(acc, [bvec, jnp.full((L,), d, jnp.int32)])
           for d in range(D)]
    for o in range(O):
      lacc = bv[o, pl.ds(0, L)]
      for d in range(D):
        lacc = lacc + pdt[d] * wv[o, d, pl.ds(0, L)]
      sig = 1.0 / (1.0 + jnp.exp(-lacc))
      plsc.store_scatter(out_v, [bvec, jnp.full((L,), o, jnp.int32)], sig)

  pltpu.sync_copy(out_v, out_hbm.at[pl.ds(base, EPW)])


def kernel(x, table, W, b):
  xf = x.reshape(-1).astype(jnp.int32)
  wrep = jnp.broadcast_to((W * (1.0 / H))[:, :, None], (O, D, L))
  brep = jnp.broadcast_to(b[:, None], (O, L))
  return _sc_kernel(xf, table, wrep, brep)


# speedup vs baseline: 2.2553x; 2.2553x over previous
import functools

import jax
import jax.numpy as jnp
from jax import lax
from jax.experimental import pallas as pl
from jax.experimental.pallas import tpu as pltpu
from jax.experimental.pallas import tpu_sc as plsc

VOCAB = 1000000
D = 32
O = 6
B = 4096
H = 200

NC = 2
NS = 16
L = 16
NW = NC * NS
EPW = B // NW
GSUB = 40
NG = H // GSUB

_mesh = plsc.VectorSubcoreMesh(
    core_axis_name="c", subcore_axis_name="s", num_cores=NC, num_subcores=NS)


@functools.partial(
    pl.kernel,
    out_type=jax.ShapeDtypeStruct((B * O,), jnp.float32),
    mesh=_mesh,
    compiler_params=pltpu.CompilerParams(
        needs_layout_passes=False, use_tc_tiling_on_sc=False),
    scratch_types=dict(
        idx_v=pltpu.VMEM((EPW * H,), jnp.int32),
        rows_a=pltpu.VMEM((H, D), jnp.float32),
        rows_b=pltpu.VMEM((H, D), jnp.float32),
        acc=pltpu.VMEM((EPW * D,), jnp.float32),
        wv=pltpu.VMEM((O, D, L), jnp.float32),
        bv=pltpu.VMEM((O, L), jnp.float32),
        out_v=pltpu.VMEM((EPW * O,), jnp.float32),
        sem_a=pltpu.SemaphoreType.DMA,
        sem_b=pltpu.SemaphoreType.DMA,
    ),
)
def _sc_kernel(x_hbm, table_hbm, w_hbm, b_hbm, out_hbm,
               idx_v, rows_a, rows_b, acc, wv, bv, out_v, sem_a, sem_b):
  wid = lax.axis_index("s") * NC + lax.axis_index("c")
  base = wid * EPW

  pltpu.sync_copy(x_hbm.at[pl.ds(base * H, EPW * H)], idx_v)
  pltpu.sync_copy(w_hbm, wv)
  pltpu.sync_copy(b_hbm, bv)

  def fire(e, buf, sem):
    for j in range(NG):
      off = pl.multiple_of(e * H + j * GSUB, 8)
      pltpu.async_copy(
          table_hbm.at[idx_v.at[pl.ds(off, GSUB)]],
          buf.at[pl.ds(j * GSUB, GSUB)], sem)

  def drain(buf, sem):
    pltpu.make_async_copy(table_hbm.at[pl.ds(0, H)], buf, sem).wait()

  zeros = jnp.zeros((L,), jnp.float32)

  def reduce(e, buf):
    def body(i, carry):
      a0, a1 = carry
      for k in range(8):
        r = i * 8 + k
        a0 = a0 + buf[r, pl.ds(0, L)]
        a1 = a1 + buf[r, pl.ds(L, L)]
      return a0, a1
    a0, a1 = lax.fori_loop(0, H // 8, body, (zeros, zeros))
    eoff = pl.multiple_of(e * D, 8)
    acc[pl.ds(eoff, L)] = a0
    acc[pl.ds(eoff + L, L)] = a1

  fire(0, rows_a, sem_a)

  def pipe(ee, _):
    e0 = ee * 2
    drain(rows_a, sem_a)
    fire(e0 + 1, rows_b, sem_b)
    reduce(e0, rows_a)
    drain(rows_b, sem_b)

    @pl.when(e0 + 2 < EPW)
    def _():
      fire(e0 + 2, rows_a, sem_a)

    reduce(e0 + 1, rows_b)
    return 0

  lax.fori_loop(0, EPW // 2, pipe, 0)

  iota = lax.iota(jnp.int32, L)
  for g in range(EPW // L):
    bvec = g * L + iota
    pdt = [plsc.load_gather(acc, [bvec * D + d]) for d in range(D)]
    for o in range(O):
      lacc = bv[o, pl.ds(0, L)]
      for d in range(D):
        lacc = lacc + pdt[d] * wv[o, d, pl.ds(0, L)]
      sig = 1.0 / (1.0 + jnp.exp(-lacc))
      plsc.store_scatter(out_v, [bvec * O + o], sig)

  pltpu.sync_copy(out_v, out_hbm.at[pl.ds(base * O, EPW * O)])


def kernel(x, table, W, b):
  xf = x.reshape(-1).astype(jnp.int32)
  wrep = jnp.broadcast_to((W * (1.0 / H))[:, :, None], (O, D, L))
  brep = jnp.broadcast_to(b[:, None], (O, L))
  return _sc_kernel(xf, table, wrep, brep).reshape(B, O)


# speedup vs baseline: 2.4929x; 1.1053x over previous
import functools

import jax
import jax.numpy as jnp
from jax import lax
from jax.experimental import pallas as pl
from jax.experimental.pallas import tpu as pltpu
from jax.experimental.pallas import tpu_sc as plsc

VOCAB = 1000000
D = 32
O = 6
B = 4096
H = 200

NC = 2
NS = 16
L = 16
NW = NC * NS
EPW = B // NW
CHUNK = 4
NCH = EPW // CHUNK
GSUB = 200
NG = CHUNK * H // GSUB

_mesh = plsc.VectorSubcoreMesh(
    core_axis_name="c", subcore_axis_name="s", num_cores=NC, num_subcores=NS)


@functools.partial(
    pl.kernel,
    out_type=jax.ShapeDtypeStruct((B * O,), jnp.float32),
    mesh=_mesh,
    compiler_params=pltpu.CompilerParams(
        needs_layout_passes=False, use_tc_tiling_on_sc=False),
    scratch_types=dict(
        idx_v=pltpu.VMEM((EPW * H,), jnp.int32),
        rows_a=pltpu.VMEM((CHUNK * H, D), jnp.float32),
        rows_b=pltpu.VMEM((CHUNK * H, D), jnp.float32),
        acc=pltpu.VMEM((EPW * D,), jnp.float32),
        wv=pltpu.VMEM((O, D, L), jnp.float32),
        bv=pltpu.VMEM((O, L), jnp.float32),
        out_v=pltpu.VMEM((EPW * O,), jnp.float32),
        sem_a=pltpu.SemaphoreType.DMA,
        sem_b=pltpu.SemaphoreType.DMA,
    ),
)
def _sc_kernel(x_hbm, table_hbm, w_hbm, b_hbm, out_hbm,
               idx_v, rows_a, rows_b, acc, wv, bv, out_v, sem_a, sem_b):
  wid = lax.axis_index("s") * NC + lax.axis_index("c")
  base = wid * EPW

  pltpu.sync_copy(x_hbm.at[pl.ds(base * H, EPW * H)], idx_v)
  pltpu.sync_copy(w_hbm, wv)
  pltpu.sync_copy(b_hbm, bv)

  def fire(c, buf, sem):
    for j in range(NG):
      off = pl.multiple_of(c * (CHUNK * H) + j * GSUB, 8)
      pltpu.async_copy(
          table_hbm.at[idx_v.at[pl.ds(off, GSUB)]],
          buf.at[pl.ds(j * GSUB, GSUB)], sem)

  def drain(buf, sem):
    pltpu.make_async_copy(table_hbm.at[pl.ds(0, CHUNK * H)], buf, sem).wait()

  zeros = jnp.zeros((L,), jnp.float32)
  zeros8 = (zeros,) * 8

  def reduce(c, buf):
    for el in range(CHUNK):
      def body(i, carry, el=el):
        c0, c1, c2, c3, c4, c5, c6, c7 = carry
        r = el * H + i * 8
        for k in range(4):
          c0 = c0 + buf[r + 2 * k, pl.ds(0, L)]
          c1 = c1 + buf[r + 2 * k, pl.ds(L, L)]
          c2 = c2 + buf[r + 2 * k + 1, pl.ds(0, L)]
          c3 = c3 + buf[r + 2 * k + 1, pl.ds(L, L)]
        return (c2, c3, c4, c5, c6, c7, c0, c1)
      ch = lax.fori_loop(0, H // 8, body, zeros8)
      a0 = (ch[0] + ch[2]) + (ch[4] + ch[6])
      a1 = (ch[1] + ch[3]) + (ch[5] + ch[7])
      eoff = pl.multiple_of((c * CHUNK + el) * D, 8)
      acc[pl.ds(eoff, L)] = a0
      acc[pl.ds(eoff + L, L)] = a1

  fire(0, rows_a, sem_a)

  def pipe(ee, _):
    c0 = ee * 2
    drain(rows_a, sem_a)
    fire(c0 + 1, rows_b, sem_b)
    reduce(c0, rows_a)
    drain(rows_b, sem_b)

    @pl.when(c0 + 2 < NCH)
    def _():
      fire(c0 + 2, rows_a, sem_a)

    reduce(c0 + 1, rows_b)
    return 0

  lax.fori_loop(0, NCH // 2, pipe, 0)

  iota = lax.iota(jnp.int32, L)
  for g in range(EPW // L):
    bvec = g * L + iota
    pdt = [plsc.load_gather(acc, [bvec * D + d]) for d in range(D)]
    for o in range(O):
      lacc = bv[o, pl.ds(0, L)]
      for d in range(D):
        lacc = lacc + pdt[d] * wv[o, d, pl.ds(0, L)]
      sig = 1.0 / (1.0 + jnp.exp(-lacc))
      plsc.store_scatter(out_v, [bvec * O + o], sig)

  pltpu.sync_copy(out_v, out_hbm.at[pl.ds(base * O, EPW * O)])


def kernel(x, table, W, b):
  xf = x.reshape(-1).astype(jnp.int32)
  wrep = jnp.broadcast_to((W * (1.0 / H))[:, :, None], (O, D, L))
  brep = jnp.broadcast_to(b[:, None], (O, L))
  return _sc_kernel(xf, table, wrep, brep).reshape(B, O)
